# baseline (speedup 1.0000x reference)
import functools

import jax
import jax.numpy as jnp
from jax import lax
from jax.experimental import pallas as pl
from jax.experimental.pallas import tpu as pltpu
from jax.experimental.pallas import tpu_sc as plsc

DIM = 256
NUM_CODES = 8192

TM = 1024
BIG = 2**30


def _argmin_body(xs_ref, cb_ref, ids_ref, c2_ref, w_ref):
    i = pl.program_id(0)

    @pl.when(i == 0)
    def _init():
        cb = cb_ref[...]
        c2_ref[...] = jnp.sum(cb * cb, axis=1)[None, :]
        colv = lax.broadcasted_iota(jnp.int32, (NUM_CODES, 1), 0)
        w_ref[...] = jnp.concatenate(
            [(colv >> 6).astype(jnp.float32),
             (colv & 63).astype(jnp.float32),
             jnp.ones((NUM_CODES, 1), jnp.float32)], axis=1)

    xs = xs_ref[...]
    s2 = lax.dot_general(
        xs, cb_ref[...], (((1,), (1,)), ((), ())),
        preferred_element_type=jnp.float32,
        precision=lax.Precision.DEFAULT)
    x2 = 0.25 * jnp.sum(xs * xs, axis=1, keepdims=True)
    d = jnp.maximum((x2 + c2_ref[...]) - s2, 0.0)

    row_min = jnp.min(d, axis=1, keepdims=True)
    onehot = jnp.where(d == row_min, 1.0, 0.0)
    sums = lax.dot_general(
        onehot, w_ref[...], (((1,), (0,)), ((), ())),
        preferred_element_type=jnp.float32,
        precision=lax.Precision.DEFAULT)
    ids_ref[...] = (64.0 * sums[:, 0:1] + sums[:, 1:2]).astype(jnp.int32)

    @pl.when(jnp.max(sums[:, 2:3]) > 1.0)
    def _tie_fallback():
        col = lax.broadcasted_iota(jnp.int32, (TM, NUM_CODES), 1)
        ids_ref[...] = jnp.min(jnp.where(d == row_min, col, jnp.int32(BIG)),
                               axis=1, keepdims=True)


def _nearest_code_ids(xs2d, codebook):
    m = xs2d.shape[0]
    return pl.pallas_call(
        _argmin_body,
        grid=(m // TM,),
        in_specs=[
            pl.BlockSpec((TM, DIM), lambda i: (i, 0)),
            pl.BlockSpec((NUM_CODES, DIM), lambda i: (0, 0)),
        ],
        out_specs=pl.BlockSpec((TM, 1), lambda i: (i, 0)),
        out_shape=jax.ShapeDtypeStruct((m, 1), jnp.int32),
        scratch_shapes=[
            pltpu.VMEM((1, NUM_CODES), jnp.float32),
            pltpu.VMEM((NUM_CODES, 3), jnp.float32),
        ],
        compiler_params=pltpu.CompilerParams(
            dimension_semantics=("arbitrary",)),
    )(xs2d, codebook)


def _make_sc_gather(n_rows):
    info = plsc.get_sparse_core_info()
    nw = info.num_cores * info.num_subcores
    per_w = n_rows // nw
    nc = info.num_cores

    @functools.partial(
        pl.kernel,
        out_type=jax.ShapeDtypeStruct((n_rows, DIM), jnp.float32),
        mesh=plsc.VectorSubcoreMesh(core_axis_name="c", subcore_axis_name="s"),
        scratch_types=[
            pltpu.VMEM((per_w,), jnp.int32),
            pltpu.VMEM((per_w, DIM), jnp.float32),
            pltpu.SemaphoreType.DMA,
        ],
    )
    def gather(table_hbm, idx_hbm, out_hbm, idx_v, rows_v, sem):
        wid = lax.axis_index("s") * nc + lax.axis_index("c")
        base = wid * per_w
        pltpu.sync_copy(idx_hbm.at[pl.ds(base, per_w)], idx_v)
        pltpu.async_copy(table_hbm.at[idx_v], rows_v, sem).wait()
        pltpu.sync_copy(rows_v, out_hbm.at[pl.ds(base, per_w)])

    return gather


def kernel(x, codebook):
    b, t, dim = x.shape
    m = b * t
    xs2d = (x + x).reshape(m, dim)
    ids2d = _nearest_code_ids(xs2d, codebook)
    ids_flat = ids2d.reshape(m)
    outputs = _make_sc_gather(m)(codebook, ids_flat)
    return (outputs.reshape(b, t, dim),
            ids2d.reshape(b, t, 1).astype(jnp.int64))


# speedup vs baseline: 1.6139x; 1.6139x over previous
import functools

import jax
import jax.numpy as jnp
from jax import lax
from jax.experimental import pallas as pl
from jax.experimental.pallas import tpu as pltpu
from jax.experimental.pallas import tpu_sc as plsc

DIM = 256
NUM_CODES = 8192

TM = 1024
BIG = 2**30


def _argmin_body(xs_ref, cb_ref, ids_ref, c2_ref, col_ref):
    i = pl.program_id(0)

    @pl.when(i == 0)
    def _init():
        cb = cb_ref[...]
        c2_ref[...] = jnp.sum(cb * cb, axis=1)[None, :]
        col_ref[...] = lax.broadcasted_iota(jnp.int32, (1, NUM_CODES), 1)

    xs = xs_ref[...]
    s2 = lax.dot_general(
        xs, cb_ref[...], (((1,), (1,)), ((), ())),
        preferred_element_type=jnp.float32,
        precision=lax.Precision.DEFAULT)
    x2 = 0.25 * jnp.sum(xs * xs, axis=1, keepdims=True)
    d = jnp.maximum((x2 + c2_ref[...]) - s2, 0.0)

    row_min = jnp.min(d, axis=1, keepdims=True)
    ids_ref[...] = jnp.min(jnp.where(d == row_min, col_ref[...],
                                     jnp.int32(BIG)),
                           axis=1, keepdims=True)


def _nearest_code_ids(xs2d, codebook):
    m = xs2d.shape[0]
    return pl.pallas_call(
        _argmin_body,
        grid=(m // TM,),
        in_specs=[
            pl.BlockSpec((TM, DIM), lambda i: (i, 0)),
            pl.BlockSpec((NUM_CODES, DIM), lambda i: (0, 0)),
        ],
        out_specs=pl.BlockSpec((TM, 1), lambda i: (i, 0)),
        out_shape=jax.ShapeDtypeStruct((m, 1), jnp.int32),
        scratch_shapes=[
            pltpu.VMEM((1, NUM_CODES), jnp.float32),
            pltpu.VMEM((1, NUM_CODES), jnp.int32),
        ],
        compiler_params=pltpu.CompilerParams(
            dimension_semantics=("arbitrary",)),
    )(xs2d, codebook)


def _make_sc_gather(n_rows):
    info = plsc.get_sparse_core_info()
    nw = info.num_cores * info.num_subcores
    per_w = n_rows // nw
    nc = info.num_cores

    @functools.partial(
        pl.kernel,
        out_type=jax.ShapeDtypeStruct((n_rows, DIM), jnp.float32),
        mesh=plsc.VectorSubcoreMesh(core_axis_name="c", subcore_axis_name="s"),
        scratch_types=[
            pltpu.VMEM((per_w,), jnp.int32),
            pltpu.VMEM((per_w, DIM), jnp.float32),
            pltpu.SemaphoreType.DMA,
        ],
    )
    def gather(table_hbm, idx_hbm, out_hbm, idx_v, rows_v, sem):
        wid = lax.axis_index("s") * nc + lax.axis_index("c")
        base = wid * per_w
        pltpu.sync_copy(idx_hbm.at[pl.ds(base, per_w)], idx_v)
        pltpu.async_copy(table_hbm.at[idx_v], rows_v, sem).wait()
        pltpu.sync_copy(rows_v, out_hbm.at[pl.ds(base, per_w)])

    return gather


def kernel(x, codebook):
    b, t, dim = x.shape
    m = b * t
    xs2d = (x + x).reshape(m, dim)
    ids2d = _nearest_code_ids(xs2d, codebook)
    ids_flat = ids2d.reshape(m)
    outputs = _make_sc_gather(m)(codebook, ids_flat)
    return (outputs.reshape(b, t, dim),
            ids2d.reshape(b, t, 1).astype(jnp.int64))
